# Initial kernel scaffold; baseline (speedup 1.0000x reference)
import jax
import jax.numpy as jnp
from jax.experimental import pallas as pl


def kernel(U, neuron_regions, eids, r_map, neuron_slot, region_emb, eid_emb, ln_g, ln_b, W1, b1, W2, b2):
    raise NotImplementedError("write your pallas kernel here")


# speedup vs baseline: 5916.3833x; 5916.3833x over previous
import jax
import jax.numpy as jnp
from jax.experimental import pallas as pl
from jax.experimental.pallas import tpu as pltpu
from functools import partial


def _decoder_body(eids_ref, regs_ref, u_ref, ns_t_ref, remb_t_ref, eemb_t_ref,
                  lng_ref, lnb_ref, w1_t_ref, b1_ref, w2p_t_ref, b2p_ref,
                  rmap_ref, out_ref):
    b = pl.program_id(0)
    T, R, Ds = u_ref.shape[1], u_ref.shape[2], u_ref.shape[3]
    N = regs_ref.shape[2]
    d_id = remb_t_ref.shape[0]
    max_regions = remb_t_ref.shape[1]
    max_eids = eemb_t_ref.shape[1]

    regs_row = regs_ref[0]
    onehot_t = (jax.lax.broadcasted_iota(jnp.int32, (max_regions, N), 0)
                == regs_row).astype(jnp.float32)

    e_t = ns_t_ref[...] + jnp.dot(remb_t_ref[...], onehot_t,
                                  preferred_element_type=jnp.float32)
    eid = eids_ref[b]
    eoh_col = (jax.lax.broadcasted_iota(jnp.int32, (max_eids, 1), 0)
               == eid).astype(jnp.float32)
    e_t = e_t + jnp.dot(eemb_t_ref[...], eoh_col,
                        preferred_element_type=jnp.float32)

    mu = jnp.mean(e_t, axis=0, keepdims=True)
    cen = e_t - mu
    var = jnp.mean(cen * cen, axis=0, keepdims=True)
    eh_t = cen * jax.lax.rsqrt(var + 1e-5) * lng_ref[...] + lnb_ref[...]

    pre = jnp.dot(w1_t_ref[...], eh_t,
                  preferred_element_type=jnp.float32) + b1_ref[...]
    h_t = 0.5 * pre * (1.0 + jax.lax.erf(pre * 0.7071067811865476))
    wb_t = jnp.dot(w2p_t_ref[...], h_t,
                   preferred_element_type=jnp.float32) + b2p_ref[...]
    w_t = wb_t[:Ds, :]
    bias_row = wb_t[Ds:Ds + 1, :]

    u2 = u_ref[...].reshape(T * R, Ds)
    s = jnp.dot(u2, w_t, preferred_element_type=jnp.float32)
    s3 = s.reshape(T, R, N)

    local_r = jnp.dot(rmap_ref[...], onehot_t,
                      preferred_element_type=jnp.float32)

    pred = jnp.zeros((T, N), jnp.float32) + bias_row
    for r in range(R):
        mask = (local_r == float(r)).astype(jnp.float32)
        pred = pred + s3[:, r, :] * mask
    out_ref[0] = pred


def kernel(U, neuron_regions, eids, r_map, neuron_slot, region_emb, eid_emb,
           ln_g, ln_b, W1, b1, W2, b2):
    B, T, R, Ds = U.shape
    N = neuron_regions.shape[1]
    d_id = neuron_slot.shape[1]
    max_regions = region_emb.shape[0]
    max_eids = eid_emb.shape[0]
    H = W1.shape[1]
    Dout = W2.shape[1]

    ns_t = neuron_slot[:N].T
    remb_t = region_emb.T
    eemb_t = eid_emb.T
    lng_col = ln_g.reshape(d_id, 1)
    lnb_col = ln_b.reshape(d_id, 1)
    w1_t = W1.T
    b1_col = b1.reshape(H, 1)
    w2p = jnp.zeros((H, H), W2.dtype).at[:, :Dout].set(W2)
    b2p = jnp.zeros((H,), b2.dtype).at[:Dout].set(b2)
    w2p_t = w2p.T
    b2p_col = b2p.reshape(H, 1)
    rmap_row = r_map.astype(jnp.float32).reshape(1, max_regions)
    regs3 = neuron_regions.reshape(B, 1, N)

    grid = (B,)
    out = pl.pallas_call(
        _decoder_body,
        grid=grid,
        in_specs=[
            pl.BlockSpec(memory_space=pltpu.SMEM),
            pl.BlockSpec((1, 1, N), lambda b: (b, 0, 0)),
            pl.BlockSpec((1, T, R, Ds), lambda b: (b, 0, 0, 0)),
            pl.BlockSpec((d_id, N), lambda b: (0, 0)),
            pl.BlockSpec((d_id, max_regions), lambda b: (0, 0)),
            pl.BlockSpec((d_id, max_eids), lambda b: (0, 0)),
            pl.BlockSpec((d_id, 1), lambda b: (0, 0)),
            pl.BlockSpec((d_id, 1), lambda b: (0, 0)),
            pl.BlockSpec((H, d_id), lambda b: (0, 0)),
            pl.BlockSpec((H, 1), lambda b: (0, 0)),
            pl.BlockSpec((H, H), lambda b: (0, 0)),
            pl.BlockSpec((H, 1), lambda b: (0, 0)),
            pl.BlockSpec((1, max_regions), lambda b: (0, 0)),
        ],
        out_specs=pl.BlockSpec((1, T, N), lambda b: (b, 0, 0)),
        out_shape=jax.ShapeDtypeStruct((B, T, N), jnp.float32),
    )(eids, regs3, U, ns_t, remb_t, eemb_t, lng_col, lnb_col,
      w1_t, b1_col, w2p_t, b2p_col, rmap_row)
    return out


# speedup vs baseline: 7142.3776x; 1.2072x over previous
import functools

import jax
import jax.numpy as jnp
from jax import lax
from jax.experimental import pallas as pl
from jax.experimental.pallas import tpu as pltpu
from jax.experimental.pallas import tpu_sc as plsc


def _sc_gather_body(regs_hbm, table_hbm, out_hbm, idx_v, rows_v, sem):
    nc = 2
    wid = lax.axis_index("s") * nc + lax.axis_index("c")
    k = idx_v.shape[0]
    base = wid * k
    pltpu.sync_copy(regs_hbm.at[pl.ds(base, k)], idx_v)
    pltpu.async_copy(table_hbm.at[idx_v], rows_v, sem).wait()
    pltpu.sync_copy(rows_v, out_hbm.at[pl.ds(base, k)])


def _tc_body(eids_ref, epl_ref, u_ref, ns_ref, eemb_ref,
             lng_ref, lnb_ref, w1_ref, b1_ref, w2p_ref, b2p_ref, out_ref):
    b = pl.program_id(0)
    T, R, Ds = u_ref.shape[1], u_ref.shape[2], u_ref.shape[3]
    N = epl_ref.shape[1]
    W = epl_ref.shape[2]
    d_id = ns_ref.shape[1]
    max_eids = eemb_ref.shape[0]
    H = w1_ref.shape[1]

    epl = epl_ref[0]

    eid = eids_ref[b]
    eoh = (lax.broadcasted_iota(jnp.int32, (1, max_eids), 1)
           == eid).astype(jnp.float32)
    eid_row = jnp.dot(eoh, eemb_ref[...], preferred_element_type=jnp.float32)
    e = epl[:, :d_id] + ns_ref[...] + eid_row

    el = (lax.broadcasted_iota(jnp.int32, (1, W), 1) == d_id).astype(jnp.float32)
    lr_row = lax.dot_general(el, epl, (((1,), (1,)), ((), ())),
                             preferred_element_type=jnp.float32)

    mu = jnp.mean(e, axis=1, keepdims=True)
    cen = e - mu
    var = jnp.mean(cen * cen, axis=1, keepdims=True)
    eh = cen * lax.rsqrt(var + 1e-5) * lng_ref[...] + lnb_ref[...]

    pre = jnp.dot(eh, w1_ref[...], preferred_element_type=jnp.float32) \
        + b1_ref[...]
    h = 0.5 * pre * (1.0 + lax.erf(pre * 0.7071067811865476))
    wb = jnp.dot(h, w2p_ref[...], preferred_element_type=jnp.float32) \
        + b2p_ref[...]
    w = wb[:, :Ds]

    u2 = u_ref[...].reshape(T * R, Ds)
    s = lax.dot_general(u2, w, (((1,), (1,)), ((), ())),
                        preferred_element_type=jnp.float32)
    s3 = s.reshape(T, R, N)

    e1 = (lax.broadcasted_iota(jnp.int32, (1, H), 1) == Ds).astype(jnp.float32)
    bias_row = lax.dot_general(e1, wb, (((1,), (1,)), ((), ())),
                               preferred_element_type=jnp.float32)

    pred = jnp.zeros((T, N), jnp.float32) + bias_row
    for r in range(R):
        mask = (lr_row == float(r)).astype(jnp.float32)
        pred = pred + s3[:, r, :] * mask
    out_ref[0] = pred


def kernel(U, neuron_regions, eids, r_map, neuron_slot, region_emb, eid_emb,
           ln_g, ln_b, W1, b1, W2, b2):
    B, T, R, Ds = U.shape
    N = neuron_regions.shape[1]
    d_id = neuron_slot.shape[1]
    max_regions = region_emb.shape[0]
    max_eids = eid_emb.shape[0]
    H = W1.shape[1]
    Dout = W2.shape[1]

    BN = B * N
    n_workers = 32
    k = BN // n_workers
    Wt = 2 * d_id
    regs_flat = neuron_regions.reshape(BN)
    table = jnp.concatenate(
        [region_emb,
         jnp.broadcast_to(r_map.astype(jnp.float32)[:, None],
                          (max_regions, Wt - d_id))], axis=1)

    mesh = plsc.VectorSubcoreMesh(core_axis_name="c", subcore_axis_name="s")
    sc_gather = functools.partial(
        pl.kernel, mesh=mesh,
        out_type=jax.ShapeDtypeStruct((BN, Wt), jnp.float32),
        scratch_types=[pltpu.VMEM((k,), jnp.int32),
                       pltpu.VMEM((k, Wt), jnp.float32),
                       pltpu.SemaphoreType.DMA],
    )(_sc_gather_body)
    epl_flat = sc_gather(regs_flat, table)
    epl = epl_flat.reshape(B, N, Wt)

    ns = neuron_slot[:N]
    lng_row = ln_g.reshape(1, d_id)
    lnb_row = ln_b.reshape(1, d_id)
    b1_row = b1.reshape(1, H)
    w2p = jnp.zeros((H, H), W2.dtype).at[:, :Dout].set(W2)
    b2p = jnp.zeros((1, H), b2.dtype).at[0, :Dout].set(b2)

    out = pl.pallas_call(
        _tc_body,
        grid=(B,),
        in_specs=[
            pl.BlockSpec(memory_space=pltpu.SMEM),
            pl.BlockSpec((1, N, Wt), lambda b: (b, 0, 0)),
            pl.BlockSpec((1, T, R, Ds), lambda b: (b, 0, 0, 0)),
            pl.BlockSpec((N, d_id), lambda b: (0, 0)),
            pl.BlockSpec((max_eids, d_id), lambda b: (0, 0)),
            pl.BlockSpec((1, d_id), lambda b: (0, 0)),
            pl.BlockSpec((1, d_id), lambda b: (0, 0)),
            pl.BlockSpec((d_id, H), lambda b: (0, 0)),
            pl.BlockSpec((1, H), lambda b: (0, 0)),
            pl.BlockSpec((H, H), lambda b: (0, 0)),
            pl.BlockSpec((1, H), lambda b: (0, 0)),
        ],
        out_specs=pl.BlockSpec((1, T, N), lambda b: (b, 0, 0)),
        out_shape=jax.ShapeDtypeStruct((B, T, N), jnp.float32),
    )(eids, epl, U, ns, eid_emb, lng_row, lnb_row, W1, b1_row, w2p, b2p)
    return out


# speedup vs baseline: 10694.3385x; 1.4973x over previous
import functools

import jax
import jax.numpy as jnp
from jax import lax
from jax.experimental import pallas as pl
from jax.experimental.pallas import tpu as pltpu
from jax.experimental.pallas import tpu_sc as plsc


def _sc_gather_body(regs_hbm, table_hbm, out_hbm, idx_v, rows_v, sem):
    nc = 2
    wid = lax.axis_index("s") * nc + lax.axis_index("c")
    k = idx_v.shape[0]
    base = wid * k
    pltpu.sync_copy(regs_hbm.at[pl.ds(base, k)], idx_v)
    pltpu.async_copy(table_hbm.at[idx_v], rows_v, sem).wait()
    pltpu.sync_copy(rows_v, out_hbm.at[pl.ds(base, k)])


def _tc_body(eids_ref, epl_ref, u_ref, ns_ref, eemb_ref,
             lng_ref, lnb_ref, w1_ref, b1_ref, w2p_ref, b2p_ref, out_ref):
    b = pl.program_id(0)
    R, T, Ds = u_ref.shape[1], u_ref.shape[2], u_ref.shape[3]
    N = epl_ref.shape[1]
    W = epl_ref.shape[2]
    d_id = ns_ref.shape[1]
    max_eids = eemb_ref.shape[0]
    H = w1_ref.shape[1]

    epl = epl_ref[0]

    eid = eids_ref[b]
    eoh = (lax.broadcasted_iota(jnp.int32, (1, max_eids), 1)
           == eid).astype(jnp.float32)
    eid_row = jnp.dot(eoh, eemb_ref[...], preferred_element_type=jnp.float32)
    e = epl[:, :d_id] + ns_ref[...] + eid_row

    el = (lax.broadcasted_iota(jnp.int32, (1, W), 1) == d_id).astype(jnp.float32)
    lr_row = lax.dot_general(el, epl, (((1,), (1,)), ((), ())),
                             preferred_element_type=jnp.float32)

    mu = jnp.mean(e, axis=1, keepdims=True)
    cen = e - mu
    var = jnp.mean(cen * cen, axis=1, keepdims=True)
    eh = cen * lax.rsqrt(var + 1e-5) * lng_ref[...] + lnb_ref[...]

    pre = jnp.dot(eh, w1_ref[...], preferred_element_type=jnp.float32) \
        + b1_ref[...]
    h = 0.5 * pre * (1.0 + lax.erf(pre * 0.7071067811865476))
    wb = jnp.dot(h, w2p_ref[...], preferred_element_type=jnp.float32) \
        + b2p_ref[...]
    w = wb[:, :Ds]

    u2 = u_ref[...].reshape(R * T, Ds)
    s = lax.dot_general(u2, w, (((1,), (1,)), ((), ())),
                        preferred_element_type=jnp.float32)
    s3 = s.reshape(R, T, N)

    e1 = (lax.broadcasted_iota(jnp.int32, (1, H), 1) == Ds).astype(jnp.float32)
    bias_row = lax.dot_general(e1, wb, (((1,), (1,)), ((), ())),
                               preferred_element_type=jnp.float32)

    pred = jnp.zeros((T, N), jnp.float32) + bias_row
    for r in range(R):
        mask = (lr_row == float(r)).astype(jnp.float32)
        pred = pred + s3[r] * mask
    out_ref[0] = pred


def kernel(U, neuron_regions, eids, r_map, neuron_slot, region_emb, eid_emb,
           ln_g, ln_b, W1, b1, W2, b2):
    B, T, R, Ds = U.shape
    N = neuron_regions.shape[1]
    d_id = neuron_slot.shape[1]
    max_regions = region_emb.shape[0]
    max_eids = eid_emb.shape[0]
    H = W1.shape[1]
    Dout = W2.shape[1]

    BN = B * N
    n_workers = 32
    k = BN // n_workers
    Wt = 2 * d_id
    regs_flat = neuron_regions.reshape(BN)
    table = jnp.concatenate(
        [region_emb,
         jnp.broadcast_to(r_map.astype(jnp.float32)[:, None],
                          (max_regions, Wt - d_id))], axis=1)

    mesh = plsc.VectorSubcoreMesh(core_axis_name="c", subcore_axis_name="s")
    sc_gather = functools.partial(
        pl.kernel, mesh=mesh,
        out_type=jax.ShapeDtypeStruct((BN, Wt), jnp.float32),
        scratch_types=[pltpu.VMEM((k,), jnp.int32),
                       pltpu.VMEM((k, Wt), jnp.float32),
                       pltpu.SemaphoreType.DMA],
    )(_sc_gather_body)
    epl_flat = sc_gather(regs_flat, table)
    epl = epl_flat.reshape(B, N, Wt)

    Ur = jnp.transpose(U, (0, 2, 1, 3))
    ns = neuron_slot[:N]
    lng_row = ln_g.reshape(1, d_id)
    lnb_row = ln_b.reshape(1, d_id)
    b1_row = b1.reshape(1, H)
    w2p = jnp.zeros((H, H), W2.dtype).at[:, :Dout].set(W2)
    b2p = jnp.zeros((1, H), b2.dtype).at[0, :Dout].set(b2)

    out = pl.pallas_call(
        _tc_body,
        grid=(B,),
        in_specs=[
            pl.BlockSpec(memory_space=pltpu.SMEM),
            pl.BlockSpec((1, N, Wt), lambda b: (b, 0, 0)),
            pl.BlockSpec((1, R, T, Ds), lambda b: (b, 0, 0, 0)),
            pl.BlockSpec((N, d_id), lambda b: (0, 0)),
            pl.BlockSpec((max_eids, d_id), lambda b: (0, 0)),
            pl.BlockSpec((1, d_id), lambda b: (0, 0)),
            pl.BlockSpec((1, d_id), lambda b: (0, 0)),
            pl.BlockSpec((d_id, H), lambda b: (0, 0)),
            pl.BlockSpec((1, H), lambda b: (0, 0)),
            pl.BlockSpec((H, H), lambda b: (0, 0)),
            pl.BlockSpec((1, H), lambda b: (0, 0)),
        ],
        out_specs=pl.BlockSpec((1, T, N), lambda b: (b, 0, 0)),
        out_shape=jax.ShapeDtypeStruct((B, T, N), jnp.float32),
    )(eids, epl, Ur, ns, eid_emb, lng_row, lnb_row, W1, b1_row, w2p, b2p)
    return out


# speedup vs baseline: 11019.4451x; 1.0304x over previous
import functools

import jax
import jax.numpy as jnp
from jax import lax
from jax.experimental import pallas as pl
from jax.experimental.pallas import tpu as pltpu
from jax.experimental.pallas import tpu_sc as plsc


def _sc_gather_body(regs_hbm, table_hbm, out_hbm, idx_v, rows_v, sem):
    nc = 2
    wid = lax.axis_index("s") * nc + lax.axis_index("c")
    k = idx_v.shape[0]
    base = wid * k
    pltpu.sync_copy(regs_hbm.at[pl.ds(base, k)], idx_v)
    pltpu.async_copy(table_hbm.at[idx_v], rows_v, sem).wait()
    pltpu.sync_copy(rows_v, out_hbm.at[pl.ds(base, k)])


def _tc_body(eids_ref, epl_ref, u_ref, ns_ref, eemb_ref,
             lng_ref, lnb_ref, w1_ref, b1_ref, w2p_ref, b2p_ref, out_ref):
    b = pl.program_id(0)
    R, T, Ds = u_ref.shape[1], u_ref.shape[2], u_ref.shape[3]
    N = epl_ref.shape[1]
    W = epl_ref.shape[2]
    d_id = ns_ref.shape[1]
    max_eids = eemb_ref.shape[0]
    H = w1_ref.shape[1]

    epl = epl_ref[0]

    eid = eids_ref[b]
    eoh = (lax.broadcasted_iota(jnp.int32, (1, max_eids), 1)
           == eid).astype(jnp.float32)
    eid_row = jnp.dot(eoh, eemb_ref[...], preferred_element_type=jnp.float32)
    e = epl[:, :d_id] + ns_ref[...] + eid_row

    el = (lax.broadcasted_iota(jnp.int32, (1, W), 1) == d_id).astype(jnp.float32)
    lr_row = lax.dot_general(el, epl, (((1,), (1,)), ((), ())),
                             preferred_element_type=jnp.float32)

    mu = jnp.mean(e, axis=1, keepdims=True)
    cen = e - mu
    var = jnp.mean(cen * cen, axis=1, keepdims=True)
    eh = cen * lax.rsqrt(var + 1e-5) * lng_ref[...] + lnb_ref[...]

    pre = jnp.dot(eh, w1_ref[...], preferred_element_type=jnp.float32) \
        + b1_ref[...]
    h = 0.5 * pre * (1.0 + lax.erf(pre * 0.7071067811865476))
    wb = jnp.dot(h, w2p_ref[...], preferred_element_type=jnp.float32) \
        + b2p_ref[...]
    w = wb[:, :Ds]

    u2 = u_ref[...].reshape(R * T, Ds).astype(jnp.bfloat16)
    s = lax.dot_general(u2, w.astype(jnp.bfloat16), (((1,), (1,)), ((), ())),
                        preferred_element_type=jnp.float32)
    s3 = s.reshape(R, T, N)

    e1 = (lax.broadcasted_iota(jnp.int32, (1, H), 1) == Ds).astype(jnp.float32)
    bias_row = lax.dot_general(e1, wb, (((1,), (1,)), ((), ())),
                               preferred_element_type=jnp.float32)

    masks = [(lr_row == float(r)).astype(jnp.float32) for r in range(R)]
    tc_rows = 8
    for t0 in range(0, T, tc_rows):
        acc = jnp.zeros((tc_rows, N), jnp.float32) + bias_row
        for r in range(R):
            acc = acc + s3[r, t0:t0 + tc_rows, :] * masks[r]
        out_ref[0, t0:t0 + tc_rows, :] = acc


def kernel(U, neuron_regions, eids, r_map, neuron_slot, region_emb, eid_emb,
           ln_g, ln_b, W1, b1, W2, b2):
    B, T, R, Ds = U.shape
    N = neuron_regions.shape[1]
    d_id = neuron_slot.shape[1]
    max_regions = region_emb.shape[0]
    max_eids = eid_emb.shape[0]
    H = W1.shape[1]
    Dout = W2.shape[1]

    BN = B * N
    n_workers = 32
    k = BN // n_workers
    Wt = 2 * d_id
    regs_flat = neuron_regions.reshape(BN)
    table = jnp.concatenate(
        [region_emb,
         jnp.broadcast_to(r_map.astype(jnp.float32)[:, None],
                          (max_regions, Wt - d_id))], axis=1)

    mesh = plsc.VectorSubcoreMesh(core_axis_name="c", subcore_axis_name="s")
    sc_gather = functools.partial(
        pl.kernel, mesh=mesh,
        out_type=jax.ShapeDtypeStruct((BN, Wt), jnp.float32),
        scratch_types=[pltpu.VMEM((k,), jnp.int32),
                       pltpu.VMEM((k, Wt), jnp.float32),
                       pltpu.SemaphoreType.DMA],
    )(_sc_gather_body)
    epl_flat = sc_gather(regs_flat, table)
    epl = epl_flat.reshape(B, N, Wt)

    Ur = jnp.transpose(U, (0, 2, 1, 3))
    ns = neuron_slot[:N]
    lng_row = ln_g.reshape(1, d_id)
    lnb_row = ln_b.reshape(1, d_id)
    b1_row = b1.reshape(1, H)
    w2p = jnp.zeros((H, H), W2.dtype).at[:, :Dout].set(W2)
    b2p = jnp.zeros((1, H), b2.dtype).at[0, :Dout].set(b2)

    out = pl.pallas_call(
        _tc_body,
        grid=(B,),
        in_specs=[
            pl.BlockSpec(memory_space=pltpu.SMEM),
            pl.BlockSpec((1, N, Wt), lambda b: (b, 0, 0)),
            pl.BlockSpec((1, R, T, Ds), lambda b: (b, 0, 0, 0)),
            pl.BlockSpec((N, d_id), lambda b: (0, 0)),
            pl.BlockSpec((max_eids, d_id), lambda b: (0, 0)),
            pl.BlockSpec((1, d_id), lambda b: (0, 0)),
            pl.BlockSpec((1, d_id), lambda b: (0, 0)),
            pl.BlockSpec((d_id, H), lambda b: (0, 0)),
            pl.BlockSpec((1, H), lambda b: (0, 0)),
            pl.BlockSpec((H, H), lambda b: (0, 0)),
            pl.BlockSpec((1, H), lambda b: (0, 0)),
        ],
        out_specs=pl.BlockSpec((1, T, N), lambda b: (b, 0, 0)),
        out_shape=jax.ShapeDtypeStruct((B, T, N), jnp.float32),
    )(eids, epl, Ur, ns, eid_emb, lng_row, lnb_row, W1, b1_row, w2p, b2p)
    return out


# speedup vs baseline: 11455.9710x; 1.0396x over previous
import functools

import jax
import jax.numpy as jnp
from jax import lax
from jax.experimental import pallas as pl
from jax.experimental.pallas import tpu as pltpu
from jax.experimental.pallas import tpu_sc as plsc


def _sc_gather_body(regs_hbm, table_hbm, out_hbm, idx_v, rows_v, sem):
    nc = 2
    wid = lax.axis_index("s") * nc + lax.axis_index("c")
    k = idx_v.shape[0]
    base = wid * k
    pltpu.sync_copy(regs_hbm.at[pl.ds(base, k)], idx_v)
    pltpu.async_copy(table_hbm.at[idx_v], rows_v, sem).wait()
    pltpu.sync_copy(rows_v, out_hbm.at[pl.ds(base, k)])


def _tc_body(eids_ref, regs_ref, epl_ref, u_ref, ns_ref, eemb_ref,
             lng_ref, lnb_ref, w1_ref, b1_ref, w2p_ref, b2p_ref,
             rmap_ref, out_ref):
    b = pl.program_id(0)
    R, T, Ds = u_ref.shape[1], u_ref.shape[2], u_ref.shape[3]
    N = epl_ref.shape[1]
    d_id = ns_ref.shape[1]
    max_regions = rmap_ref.shape[1]
    max_eids = eemb_ref.shape[0]
    H = w1_ref.shape[1]

    eid = eids_ref[b]
    eoh = (lax.broadcasted_iota(jnp.int32, (1, max_eids), 1)
           == eid).astype(jnp.float32)
    eid_row = jnp.dot(eoh, eemb_ref[...], preferred_element_type=jnp.float32)
    e = epl_ref[0] + ns_ref[...] + eid_row

    regs_row = regs_ref[0]
    onehot_t = (lax.broadcasted_iota(jnp.int32, (max_regions, N), 0)
                == regs_row).astype(jnp.float32)
    lr_row = jnp.dot(rmap_ref[...], onehot_t,
                     preferred_element_type=jnp.float32)

    mu = jnp.mean(e, axis=1, keepdims=True)
    cen = e - mu
    var = jnp.mean(cen * cen, axis=1, keepdims=True)
    eh = cen * lax.rsqrt(var + 1e-5) * lng_ref[...] + lnb_ref[...]

    pre = jnp.dot(eh.astype(jnp.bfloat16), w1_ref[...].astype(jnp.bfloat16),
                  preferred_element_type=jnp.float32) \
        + b1_ref[...]
    h = 0.5 * pre * (1.0 + lax.erf(pre * 0.7071067811865476))
    wb = jnp.dot(h.astype(jnp.bfloat16), w2p_ref[...].astype(jnp.bfloat16),
                 preferred_element_type=jnp.float32) \
        + b2p_ref[...]
    w = wb[:, :Ds]

    u2 = u_ref[...].reshape(R * T, Ds).astype(jnp.bfloat16)
    s = lax.dot_general(u2, w.astype(jnp.bfloat16), (((1,), (1,)), ((), ())),
                        preferred_element_type=jnp.float32)
    s3 = s.reshape(R, T, N)

    e1 = (lax.broadcasted_iota(jnp.int32, (1, H), 1) == Ds).astype(jnp.float32)
    bias_row = lax.dot_general(e1, wb, (((1,), (1,)), ((), ())),
                               preferred_element_type=jnp.float32)

    masks = [(lr_row == float(r)).astype(jnp.float32) for r in range(R)]
    tc_rows = 8
    for t0 in range(0, T, tc_rows):
        acc = jnp.zeros((tc_rows, N), jnp.float32) + bias_row
        for r in range(R):
            acc = acc + s3[r, t0:t0 + tc_rows, :] * masks[r]
        out_ref[0, t0:t0 + tc_rows, :] = acc


def kernel(U, neuron_regions, eids, r_map, neuron_slot, region_emb, eid_emb,
           ln_g, ln_b, W1, b1, W2, b2):
    B, T, R, Ds = U.shape
    N = neuron_regions.shape[1]
    d_id = neuron_slot.shape[1]
    max_regions = region_emb.shape[0]
    max_eids = eid_emb.shape[0]
    H = W1.shape[1]
    Dout = W2.shape[1]

    BN = B * N
    n_workers = 32
    k = BN // n_workers
    Wt = d_id
    regs_flat = neuron_regions.reshape(BN)

    mesh = plsc.VectorSubcoreMesh(core_axis_name="c", subcore_axis_name="s")
    sc_gather = functools.partial(
        pl.kernel, mesh=mesh,
        out_type=jax.ShapeDtypeStruct((BN, Wt), jnp.float32),
        scratch_types=[pltpu.VMEM((k,), jnp.int32),
                       pltpu.VMEM((k, Wt), jnp.float32),
                       pltpu.SemaphoreType.DMA],
    )(_sc_gather_body)
    epl_flat = sc_gather(regs_flat, region_emb)
    epl = epl_flat.reshape(B, N, Wt)

    Ur = jnp.transpose(U, (0, 2, 1, 3))
    ns = neuron_slot[:N]
    lng_row = ln_g.reshape(1, d_id)
    lnb_row = ln_b.reshape(1, d_id)
    b1_row = b1.reshape(1, H)
    w2p = jnp.zeros((H, H), W2.dtype).at[:, :Dout].set(W2)
    b2p = jnp.zeros((1, H), b2.dtype).at[0, :Dout].set(b2)
    rmap_row = r_map.astype(jnp.float32).reshape(1, max_regions)
    regs3 = neuron_regions.reshape(B, 1, N)

    out = pl.pallas_call(
        _tc_body,
        grid=(B,),
        in_specs=[
            pl.BlockSpec(memory_space=pltpu.SMEM),
            pl.BlockSpec((1, 1, N), lambda b: (b, 0, 0)),
            pl.BlockSpec((1, N, Wt), lambda b: (b, 0, 0)),
            pl.BlockSpec((1, R, T, Ds), lambda b: (b, 0, 0, 0)),
            pl.BlockSpec((N, d_id), lambda b: (0, 0)),
            pl.BlockSpec((max_eids, d_id), lambda b: (0, 0)),
            pl.BlockSpec((1, d_id), lambda b: (0, 0)),
            pl.BlockSpec((1, d_id), lambda b: (0, 0)),
            pl.BlockSpec((d_id, H), lambda b: (0, 0)),
            pl.BlockSpec((1, H), lambda b: (0, 0)),
            pl.BlockSpec((H, H), lambda b: (0, 0)),
            pl.BlockSpec((1, H), lambda b: (0, 0)),
            pl.BlockSpec((1, max_regions), lambda b: (0, 0)),
        ],
        out_specs=pl.BlockSpec((1, T, N), lambda b: (b, 0, 0)),
        out_shape=jax.ShapeDtypeStruct((B, T, N), jnp.float32),
    )(eids, regs3, epl, Ur, ns, eid_emb, lng_row, lnb_row, W1, b1_row,
      w2p, b2p, rmap_row)
    return out


# speedup vs baseline: 11516.3127x; 1.0053x over previous
import functools

import jax
import jax.numpy as jnp
from jax import lax
from jax.experimental import pallas as pl
from jax.experimental.pallas import tpu as pltpu
from jax.experimental.pallas import tpu_sc as plsc


def _sc_gather_body(regs_hbm, table_hbm, out_hbm, idx_v, rows_v, sem):
    nc = 2
    wid = lax.axis_index("s") * nc + lax.axis_index("c")
    k = idx_v.shape[0]
    base = wid * k
    pltpu.sync_copy(regs_hbm.at[pl.ds(base, k)], idx_v)
    pltpu.async_copy(table_hbm.at[idx_v], rows_v, sem).wait()
    pltpu.sync_copy(rows_v, out_hbm.at[pl.ds(base, k)])


def _tc_body(eids_ref, regs_ref, epl_ref, u_ref, ns_ref, eemb_ref,
             lng_ref, lnb_ref, w1_ref, b1_ref, w2_ref, b2_ref,
             rmap_ref, out_ref):
    b = pl.program_id(0)
    R, T, Ds = u_ref.shape[1], u_ref.shape[2], u_ref.shape[3]
    N = epl_ref.shape[1]
    d_id = ns_ref.shape[1]
    max_regions = rmap_ref.shape[1]
    max_eids = eemb_ref.shape[0]
    H = w1_ref.shape[1]

    eid = eids_ref[b]
    eoh = (lax.broadcasted_iota(jnp.int32, (1, max_eids), 1)
           == eid).astype(jnp.float32)
    eid_row = jnp.dot(eoh, eemb_ref[...], preferred_element_type=jnp.float32)
    e = epl_ref[0] + ns_ref[...] + eid_row

    regs_row = regs_ref[0]
    onehot_t = (lax.broadcasted_iota(jnp.int32, (max_regions, N), 0)
                == regs_row).astype(jnp.float32)
    lr_row = jnp.dot(rmap_ref[...], onehot_t,
                     preferred_element_type=jnp.float32)

    mu = jnp.mean(e, axis=1, keepdims=True)
    cen = e - mu
    var = jnp.mean(cen * cen, axis=1, keepdims=True)
    eh = cen * lax.rsqrt(var + 1e-5) * lng_ref[...] + lnb_ref[...]

    pre = jnp.dot(eh.astype(jnp.bfloat16), w1_ref[...].astype(jnp.bfloat16),
                  preferred_element_type=jnp.float32) \
        + b1_ref[...]
    h = 0.5 * pre * (1.0 + lax.erf(pre * 0.7071067811865476))
    wb = jnp.dot(h.astype(jnp.bfloat16), w2_ref[...].astype(jnp.bfloat16),
                 preferred_element_type=jnp.float32) \
        + b2_ref[...]
    w = wb[:, :Ds]

    u2 = u_ref[...].reshape(R * T, Ds).astype(jnp.bfloat16)
    s = lax.dot_general(u2, w.astype(jnp.bfloat16), (((1,), (1,)), ((), ())),
                        preferred_element_type=jnp.float32)
    s3 = s.reshape(R, T, N)

    Dout = wb.shape[1]
    e1 = (lax.broadcasted_iota(jnp.int32, (1, Dout), 1) == Ds).astype(jnp.float32)
    bias_row = lax.dot_general(e1, wb, (((1,), (1,)), ((), ())),
                               preferred_element_type=jnp.float32)

    masks = [(lr_row == float(r)).astype(jnp.float32) for r in range(R)]
    tc_rows = 8
    for t0 in range(0, T, tc_rows):
        acc = jnp.zeros((tc_rows, N), jnp.float32) + bias_row
        for r in range(R):
            acc = acc + s3[r, t0:t0 + tc_rows, :] * masks[r]
        out_ref[0, t0:t0 + tc_rows, :] = acc


def kernel(U, neuron_regions, eids, r_map, neuron_slot, region_emb, eid_emb,
           ln_g, ln_b, W1, b1, W2, b2):
    B, T, R, Ds = U.shape
    N = neuron_regions.shape[1]
    d_id = neuron_slot.shape[1]
    max_regions = region_emb.shape[0]
    max_eids = eid_emb.shape[0]
    H = W1.shape[1]
    Dout = W2.shape[1]

    BN = B * N
    n_workers = 32
    k = BN // n_workers
    Wt = d_id
    regs_flat = neuron_regions.reshape(BN)

    mesh = plsc.VectorSubcoreMesh(core_axis_name="c", subcore_axis_name="s")
    sc_gather = functools.partial(
        pl.kernel, mesh=mesh,
        out_type=jax.ShapeDtypeStruct((BN, Wt), jnp.float32),
        scratch_types=[pltpu.VMEM((k,), jnp.int32),
                       pltpu.VMEM((k, Wt), jnp.float32),
                       pltpu.SemaphoreType.DMA],
    )(_sc_gather_body)
    epl_flat = sc_gather(regs_flat, region_emb)
    epl = epl_flat.reshape(B, N, Wt)

    Ur = jnp.transpose(U, (0, 2, 1, 3))
    ns = neuron_slot[:N]
    lng_row = ln_g.reshape(1, d_id)
    lnb_row = ln_b.reshape(1, d_id)
    b1_row = b1.reshape(1, H)
    b2_row = b2.reshape(1, Dout)
    rmap_row = r_map.astype(jnp.float32).reshape(1, max_regions)
    regs3 = neuron_regions.reshape(B, 1, N)

    out = pl.pallas_call(
        _tc_body,
        grid=(B,),
        in_specs=[
            pl.BlockSpec(memory_space=pltpu.SMEM),
            pl.BlockSpec((1, 1, N), lambda b: (b, 0, 0)),
            pl.BlockSpec((1, N, Wt), lambda b: (b, 0, 0)),
            pl.BlockSpec((1, R, T, Ds), lambda b: (b, 0, 0, 0)),
            pl.BlockSpec((N, d_id), lambda b: (0, 0)),
            pl.BlockSpec((max_eids, d_id), lambda b: (0, 0)),
            pl.BlockSpec((1, d_id), lambda b: (0, 0)),
            pl.BlockSpec((1, d_id), lambda b: (0, 0)),
            pl.BlockSpec((d_id, H), lambda b: (0, 0)),
            pl.BlockSpec((1, H), lambda b: (0, 0)),
            pl.BlockSpec((H, Dout), lambda b: (0, 0)),
            pl.BlockSpec((1, Dout), lambda b: (0, 0)),
            pl.BlockSpec((1, max_regions), lambda b: (0, 0)),
        ],
        out_specs=pl.BlockSpec((1, T, N), lambda b: (b, 0, 0)),
        out_shape=jax.ShapeDtypeStruct((B, T, N), jnp.float32),
    )(eids, regs3, epl, Ur, ns, eid_emb, lng_row, lnb_row, W1, b1_row,
      W2, b2_row, rmap_row)
    return out


# speedup vs baseline: 11997.5032x; 1.0418x over previous
import functools

import jax
import jax.numpy as jnp
from jax import lax
from jax.experimental import pallas as pl
from jax.experimental.pallas import tpu as pltpu
from jax.experimental.pallas import tpu_sc as plsc


def _sc_gather_body(regs_hbm, table_hbm, out_hbm, idx_v, rows_v, sem):
    nc = 2
    wid = lax.axis_index("s") * nc + lax.axis_index("c")
    k = idx_v.shape[0]
    base = wid * k
    pltpu.sync_copy(regs_hbm.at[pl.ds(base, k)], idx_v)
    pltpu.async_copy(table_hbm.at[idx_v], rows_v, sem).wait()
    pltpu.sync_copy(rows_v, out_hbm.at[pl.ds(base, k)])


def _tc_body(eids_ref, regs_ref, epl_ref, u_ref, ns_ref, eemb_ref,
             lng_ref, lnb_ref, w1_ref, b1_ref, w2_ref, b2_ref,
             rmap_ref, out_ref):
    B = epl_ref.shape[0]
    R, T, Ds = u_ref.shape[1], u_ref.shape[2], u_ref.shape[3]
    N = epl_ref.shape[1]
    d_id = ns_ref.shape[1]
    max_regions = rmap_ref.shape[1]
    max_eids = eemb_ref.shape[0]
    H = w1_ref.shape[1]
    BN = B * N

    eoh = jnp.concatenate(
        [(lax.broadcasted_iota(jnp.int32, (1, max_eids), 1)
          == eids_ref[bb]).astype(jnp.float32) for bb in range(B)], axis=0)
    eid_rows = jnp.dot(eoh, eemb_ref[...],
                       preferred_element_type=jnp.float32)
    e3 = epl_ref[...] + ns_ref[...][None] + eid_rows[:, None, :]
    e = e3.reshape(BN, d_id)

    mu = jnp.mean(e, axis=1, keepdims=True)
    cen = e - mu
    var = jnp.mean(cen * cen, axis=1, keepdims=True)
    eh = cen * lax.rsqrt(var + 1e-5) * lng_ref[...] + lnb_ref[...]

    pre = jnp.dot(eh.astype(jnp.bfloat16), w1_ref[...].astype(jnp.bfloat16),
                  preferred_element_type=jnp.float32) \
        + b1_ref[...]
    h = 0.5 * pre * (1.0 + lax.erf(pre * 0.7071067811865476))
    wb = jnp.dot(h.astype(jnp.bfloat16), w2_ref[...].astype(jnp.bfloat16),
                 preferred_element_type=jnp.float32) \
        + b2_ref[...]
    w16 = wb[:, :Ds].astype(jnp.bfloat16)

    Dout = wb.shape[1]
    e1 = (lax.broadcasted_iota(jnp.int32, (1, Dout), 1) == Ds).astype(jnp.float32)
    bias_full = lax.dot_general(e1, wb, (((1,), (1,)), ((), ())),
                                preferred_element_type=jnp.float32)

    tc_rows = 8
    for b in range(B):
        regs_row = regs_ref[b]
        onehot_t = (lax.broadcasted_iota(jnp.int32, (max_regions, N), 0)
                    == regs_row).astype(jnp.float32)
        lr_row = jnp.dot(rmap_ref[...], onehot_t,
                         preferred_element_type=jnp.float32)

        u2 = u_ref[b].reshape(R * T, Ds).astype(jnp.bfloat16)
        s = lax.dot_general(u2, w16[b * N:(b + 1) * N, :],
                            (((1,), (1,)), ((), ())),
                            preferred_element_type=jnp.float32)
        s3 = s.reshape(R, T, N)
        bias_row = bias_full[:, b * N:(b + 1) * N]

        masks = [(lr_row == float(r)).astype(jnp.float32) for r in range(R)]
        for t0 in range(0, T, tc_rows):
            acc = jnp.zeros((tc_rows, N), jnp.float32) + bias_row
            for r in range(R):
                acc = acc + s3[r, t0:t0 + tc_rows, :] * masks[r]
            out_ref[b, t0:t0 + tc_rows, :] = acc


def kernel(U, neuron_regions, eids, r_map, neuron_slot, region_emb, eid_emb,
           ln_g, ln_b, W1, b1, W2, b2):
    B, T, R, Ds = U.shape
    N = neuron_regions.shape[1]
    d_id = neuron_slot.shape[1]
    max_regions = region_emb.shape[0]
    max_eids = eid_emb.shape[0]
    H = W1.shape[1]
    Dout = W2.shape[1]

    BN = B * N
    n_workers = 32
    k = BN // n_workers
    Wt = d_id
    regs_flat = neuron_regions.reshape(BN)

    mesh = plsc.VectorSubcoreMesh(core_axis_name="c", subcore_axis_name="s")
    sc_gather = functools.partial(
        pl.kernel, mesh=mesh,
        out_type=jax.ShapeDtypeStruct((BN, Wt), jnp.float32),
        scratch_types=[pltpu.VMEM((k,), jnp.int32),
                       pltpu.VMEM((k, Wt), jnp.float32),
                       pltpu.SemaphoreType.DMA],
    )(_sc_gather_body)
    epl_flat = sc_gather(regs_flat, region_emb)
    epl = epl_flat.reshape(B, N, Wt)

    Ur = jnp.transpose(U, (0, 2, 1, 3))
    ns = neuron_slot[:N]
    lng_row = ln_g.reshape(1, d_id)
    lnb_row = ln_b.reshape(1, d_id)
    b1_row = b1.reshape(1, H)
    b2_row = b2.reshape(1, Dout)
    rmap_row = r_map.astype(jnp.float32).reshape(1, max_regions)
    regs3 = neuron_regions.reshape(B, 1, N)

    out = pl.pallas_call(
        _tc_body,
        in_specs=[
            pl.BlockSpec(memory_space=pltpu.SMEM),
            pl.BlockSpec((B, 1, N), lambda: (0, 0, 0)),
            pl.BlockSpec((B, N, Wt), lambda: (0, 0, 0)),
            pl.BlockSpec((B, R, T, Ds), lambda: (0, 0, 0, 0)),
            pl.BlockSpec((N, d_id), lambda: (0, 0)),
            pl.BlockSpec((max_eids, d_id), lambda: (0, 0)),
            pl.BlockSpec((1, d_id), lambda: (0, 0)),
            pl.BlockSpec((1, d_id), lambda: (0, 0)),
            pl.BlockSpec((d_id, H), lambda: (0, 0)),
            pl.BlockSpec((1, H), lambda: (0, 0)),
            pl.BlockSpec((H, Dout), lambda: (0, 0)),
            pl.BlockSpec((1, Dout), lambda: (0, 0)),
            pl.BlockSpec((1, max_regions), lambda: (0, 0)),
        ],
        out_specs=pl.BlockSpec((B, T, N), lambda: (0, 0, 0)),
        out_shape=jax.ShapeDtypeStruct((B, T, N), jnp.float32),
    )(eids, regs3, epl, Ur, ns, eid_emb, lng_row, lnb_row, W1, b1_row,
      W2, b2_row, rmap_row)
    return out


# speedup vs baseline: 12069.9507x; 1.0060x over previous
import functools

import jax
import jax.numpy as jnp
from jax import lax
from jax.experimental import pallas as pl
from jax.experimental.pallas import tpu as pltpu
from jax.experimental.pallas import tpu_sc as plsc


def _sc_gather_body(regs_hbm, table_hbm, out_hbm, idx_v, rows_v, sem):
    nc = 2
    wid = lax.axis_index("s") * nc + lax.axis_index("c")
    k = idx_v.shape[0]
    base = wid * k
    pltpu.sync_copy(regs_hbm.at[pl.ds(base, k)], idx_v)
    pltpu.async_copy(table_hbm.at[idx_v], rows_v, sem).wait()
    pltpu.sync_copy(rows_v, out_hbm.at[pl.ds(base, k)])


def _tc_body(eids_ref, regs_ref, epl_ref, u_ref, ns_ref, eemb_ref,
             lng_ref, lnb_ref, w1_ref, b1_ref, w2_ref, b2_ref,
             rmap_ref, out_ref):
    B = epl_ref.shape[0]
    T, R, Ds = u_ref.shape[1], u_ref.shape[2], u_ref.shape[3]
    N = epl_ref.shape[1]
    d_id = ns_ref.shape[1]
    max_regions = rmap_ref.shape[1]
    max_eids = eemb_ref.shape[0]
    H = w1_ref.shape[1]
    BN = B * N

    eoh = jnp.concatenate(
        [(lax.broadcasted_iota(jnp.int32, (1, max_eids), 1)
          == eids_ref[bb]).astype(jnp.float32) for bb in range(B)], axis=0)
    eid_rows = jnp.dot(eoh, eemb_ref[...],
                       preferred_element_type=jnp.float32)
    e3 = epl_ref[...] + ns_ref[...][None] + eid_rows[:, None, :]
    e = e3.reshape(BN, d_id)

    mu = jnp.mean(e, axis=1, keepdims=True)
    cen = e - mu
    var = jnp.mean(cen * cen, axis=1, keepdims=True)
    eh = cen * lax.rsqrt(var + 1e-5) * lng_ref[...] + lnb_ref[...]

    pre = jnp.dot(eh.astype(jnp.bfloat16), w1_ref[...].astype(jnp.bfloat16),
                  preferred_element_type=jnp.float32) \
        + b1_ref[...]
    h = 0.5 * pre * (1.0 + lax.erf(pre * 0.7071067811865476))
    wb = jnp.dot(h.astype(jnp.bfloat16), w2_ref[...].astype(jnp.bfloat16),
                 preferred_element_type=jnp.float32) \
        + b2_ref[...]
    w16 = wb[:, :Ds].astype(jnp.bfloat16)

    Dout = wb.shape[1]
    e1 = (lax.broadcasted_iota(jnp.int32, (1, Dout), 1) == Ds).astype(jnp.float32)
    bias_full = lax.dot_general(e1, wb, (((1,), (1,)), ((), ())),
                                preferred_element_type=jnp.float32)

    tc_rows = 8
    for b in range(B):
        regs_row = regs_ref[b]
        onehot_t = (lax.broadcasted_iota(jnp.int32, (max_regions, N), 0)
                    == regs_row).astype(jnp.float32)
        lr_row = jnp.dot(rmap_ref[...], onehot_t,
                         preferred_element_type=jnp.float32)

        u2 = jnp.transpose(u_ref[b], (1, 0, 2)).reshape(
            R * T, Ds).astype(jnp.bfloat16)
        s = lax.dot_general(u2, w16[b * N:(b + 1) * N, :],
                            (((1,), (1,)), ((), ())),
                            preferred_element_type=jnp.float32)
        s3 = s.reshape(R, T, N)
        bias_row = bias_full[:, b * N:(b + 1) * N]

        masks = [(lr_row == float(r)).astype(jnp.float32) for r in range(R)]
        for t0 in range(0, T, tc_rows):
            acc = jnp.zeros((tc_rows, N), jnp.float32) + bias_row
            for r in range(R):
                acc = acc + s3[r, t0:t0 + tc_rows, :] * masks[r]
            out_ref[b, t0:t0 + tc_rows, :] = acc


def kernel(U, neuron_regions, eids, r_map, neuron_slot, region_emb, eid_emb,
           ln_g, ln_b, W1, b1, W2, b2):
    B, T, R, Ds = U.shape
    N = neuron_regions.shape[1]
    d_id = neuron_slot.shape[1]
    max_regions = region_emb.shape[0]
    max_eids = eid_emb.shape[0]
    H = W1.shape[1]
    Dout = W2.shape[1]

    BN = B * N
    n_workers = 32
    k = BN // n_workers
    Wt = d_id
    regs_flat = neuron_regions.reshape(BN)

    mesh = plsc.VectorSubcoreMesh(core_axis_name="c", subcore_axis_name="s")
    sc_gather = functools.partial(
        pl.kernel, mesh=mesh,
        out_type=jax.ShapeDtypeStruct((BN, Wt), jnp.float32),
        scratch_types=[pltpu.VMEM((k,), jnp.int32),
                       pltpu.VMEM((k, Wt), jnp.float32),
                       pltpu.SemaphoreType.DMA],
    )(_sc_gather_body)
    epl_flat = sc_gather(regs_flat, region_emb)
    epl = epl_flat.reshape(B, N, Wt)

    ns = neuron_slot[:N]
    lng_row = ln_g.reshape(1, d_id)
    lnb_row = ln_b.reshape(1, d_id)
    b1_row = b1.reshape(1, H)
    b2_row = b2.reshape(1, Dout)
    rmap_row = r_map.astype(jnp.float32).reshape(1, max_regions)
    regs3 = neuron_regions.reshape(B, 1, N)

    out = pl.pallas_call(
        _tc_body,
        in_specs=[
            pl.BlockSpec(memory_space=pltpu.SMEM),
            pl.BlockSpec((B, 1, N), lambda: (0, 0, 0)),
            pl.BlockSpec((B, N, Wt), lambda: (0, 0, 0)),
            pl.BlockSpec((B, T, R, Ds), lambda: (0, 0, 0, 0)),
            pl.BlockSpec((N, d_id), lambda: (0, 0)),
            pl.BlockSpec((max_eids, d_id), lambda: (0, 0)),
            pl.BlockSpec((1, d_id), lambda: (0, 0)),
            pl.BlockSpec((1, d_id), lambda: (0, 0)),
            pl.BlockSpec((d_id, H), lambda: (0, 0)),
            pl.BlockSpec((1, H), lambda: (0, 0)),
            pl.BlockSpec((H, Dout), lambda: (0, 0)),
            pl.BlockSpec((1, Dout), lambda: (0, 0)),
            pl.BlockSpec((1, max_regions), lambda: (0, 0)),
        ],
        out_specs=pl.BlockSpec((B, T, N), lambda: (0, 0, 0)),
        out_shape=jax.ShapeDtypeStruct((B, T, N), jnp.float32),
    )(eids, regs3, epl, U, ns, eid_emb, lng_row, lnb_row, W1, b1_row,
      W2, b2_row, rmap_row)
    return out


# speedup vs baseline: 12284.5963x; 1.0178x over previous
import functools

import jax
import jax.numpy as jnp
from jax import lax
from jax.experimental import pallas as pl
from jax.experimental.pallas import tpu as pltpu
from jax.experimental.pallas import tpu_sc as plsc


def _sc_gather_body(regs_hbm, table_hbm, out_hbm, idx_v, rows_v, sem):
    nc = 2
    wid = lax.axis_index("s") * nc + lax.axis_index("c")
    k = idx_v.shape[0]
    base = wid * k
    pltpu.sync_copy(regs_hbm.at[pl.ds(base, k)], idx_v)
    pltpu.async_copy(table_hbm.at[idx_v], rows_v, sem).wait()
    pltpu.sync_copy(rows_v, out_hbm.at[pl.ds(base, k)])


def _tc_body(eids_ref, regs_ref, epl_ref, u_ref, ns_ref, eemb_ref,
             lng_ref, lnb_ref, w1_ref, b1_ref, w2_ref, b2_ref,
             rmap_ref, out_ref):
    B = epl_ref.shape[0]
    T, R, Ds = u_ref.shape[1], u_ref.shape[2], u_ref.shape[3]
    N = epl_ref.shape[1]
    d_id = ns_ref.shape[1]
    max_regions = rmap_ref.shape[1]
    max_eids = eemb_ref.shape[0]
    H = w1_ref.shape[1]
    BN = B * N

    eoh = jnp.concatenate(
        [(lax.broadcasted_iota(jnp.int32, (1, max_eids), 1)
          == eids_ref[bb]).astype(jnp.float32) for bb in range(B)], axis=0)
    eid_rows = jnp.dot(eoh, eemb_ref[...],
                       preferred_element_type=jnp.float32)
    e3 = epl_ref[...] + ns_ref[...][None] + eid_rows[:, None, :]
    e = e3.reshape(BN, d_id)

    mu = jnp.mean(e, axis=1, keepdims=True)
    cen = e - mu
    var = jnp.mean(cen * cen, axis=1, keepdims=True)
    eh = cen * lax.rsqrt(var + 1e-5) * lng_ref[...] + lnb_ref[...]

    pre = jnp.dot(eh.astype(jnp.bfloat16), w1_ref[...].astype(jnp.bfloat16),
                  preferred_element_type=jnp.float32) \
        + b1_ref[...]
    h = 0.5 * pre * (1.0 + lax.erf(pre * 0.7071067811865476))
    wb = jnp.dot(h.astype(jnp.bfloat16), w2_ref[...].astype(jnp.bfloat16),
                 preferred_element_type=jnp.float32) \
        + b2_ref[...]
    w16 = wb[:, :Ds].astype(jnp.bfloat16)

    Dout = wb.shape[1]
    e1 = (lax.broadcasted_iota(jnp.int32, (1, Dout), 1) == Ds).astype(jnp.float32)
    bias_full = lax.dot_general(e1, wb, (((1,), (1,)), ((), ())),
                                preferred_element_type=jnp.float32)

    tc_rows = 8
    for b in range(B):
        regs_row = regs_ref[b]
        onehot_t = (lax.broadcasted_iota(jnp.int32, (max_regions, N), 0)
                    == regs_row).astype(jnp.float32)
        lr_row = jnp.dot(rmap_ref[...], onehot_t,
                         preferred_element_type=jnp.float32)

        u2 = jnp.transpose(u_ref[b], (1, 0, 2)).reshape(
            R * T, Ds).astype(jnp.bfloat16)
        s = lax.dot_general(u2, w16[b * N:(b + 1) * N, :],
                            (((1,), (1,)), ((), ())),
                            preferred_element_type=jnp.float32)
        s3 = s.reshape(R, T, N)
        bias_row = bias_full[:, b * N:(b + 1) * N]

        masks = [(lr_row == float(r)).astype(jnp.float32) for r in range(R)]
        for t0 in range(0, T, tc_rows):
            acc = jnp.zeros((tc_rows, N), jnp.float32) + bias_row
            for r in range(R):
                acc = acc + s3[r, t0:t0 + tc_rows, :] * masks[r]
            out_ref[b, t0:t0 + tc_rows, :] = acc


def kernel(U, neuron_regions, eids, r_map, neuron_slot, region_emb, eid_emb,
           ln_g, ln_b, W1, b1, W2, b2):
    B, T, R, Ds = U.shape
    N = neuron_regions.shape[1]
    d_id = neuron_slot.shape[1]
    max_regions = region_emb.shape[0]
    max_eids = eid_emb.shape[0]
    H = W1.shape[1]
    Dout = W2.shape[1]

    BN = B * N
    n_workers = 32
    k = BN // n_workers
    Wt = d_id
    regs_flat = neuron_regions.reshape(BN)

    mesh = plsc.VectorSubcoreMesh(core_axis_name="c", subcore_axis_name="s")
    sc_gather = functools.partial(
        pl.kernel, mesh=mesh,
        out_type=jax.ShapeDtypeStruct((BN, Wt), jnp.float32),
        scratch_types=[pltpu.VMEM((k,), jnp.int32),
                       pltpu.VMEM((k, Wt), jnp.float32),
                       pltpu.SemaphoreType.DMA],
    )(_sc_gather_body)
    epl_flat = sc_gather(regs_flat, region_emb)
    epl = epl_flat.reshape(B, N, Wt)

    lng_row = ln_g.reshape(1, d_id)
    lnb_row = ln_b.reshape(1, d_id)
    b1_row = b1.reshape(1, H)
    b2_row = b2.reshape(1, Dout)
    rmap_row = r_map.astype(jnp.float32).reshape(1, max_regions)
    regs3 = neuron_regions.reshape(B, 1, N)

    out = pl.pallas_call(
        _tc_body,
        grid=(1,),
        in_specs=[
            pl.BlockSpec(memory_space=pltpu.SMEM),
            pl.BlockSpec((B, 1, N), lambda i: (0, 0, 0)),
            pl.BlockSpec((B, N, Wt), lambda i: (0, 0, 0)),
            pl.BlockSpec((B, T, R, Ds), lambda i: (0, 0, 0, 0)),
            pl.BlockSpec((N, d_id), lambda i: (0, 0)),
            pl.BlockSpec((max_eids, d_id), lambda i: (0, 0)),
            pl.BlockSpec((1, d_id), lambda i: (0, 0)),
            pl.BlockSpec((1, d_id), lambda i: (0, 0)),
            pl.BlockSpec((d_id, H), lambda i: (0, 0)),
            pl.BlockSpec((1, H), lambda i: (0, 0)),
            pl.BlockSpec((H, Dout), lambda i: (0, 0)),
            pl.BlockSpec((1, Dout), lambda i: (0, 0)),
            pl.BlockSpec((1, max_regions), lambda i: (0, 0)),
        ],
        out_specs=pl.BlockSpec((B, T, N), lambda i: (0, 0, 0)),
        out_shape=jax.ShapeDtypeStruct((B, T, N), jnp.float32),
    )(eids, regs3, epl, U, neuron_slot, eid_emb, lng_row, lnb_row, W1, b1_row,
      W2, b2_row, rmap_row)
    return out
